# Initial kernel scaffold; baseline (speedup 1.0000x reference)
import jax
import jax.numpy as jnp
from jax.experimental import pallas as pl


def kernel(x, edge_index, edge_attr, Wr1, Wroot1, b1, Wr2, Wroot2, b2):
    raise NotImplementedError("write your pallas kernel here")


# speedup vs baseline: 168.0509x; 168.0509x over previous
import functools

import jax
import jax.numpy as jnp
from jax import lax
from jax.experimental import pallas as pl
from jax.experimental.pallas import tpu as pltpu
from jax.experimental.pallas import tpu_sc as plsc

N = 10000
E = 320000
D = 128
R = 4
NC = 2
NS = 16
CW = 32
NCH = D // CW
SUB = 80
TILE = 25
TILE_A = 10
SEGROWS = R * N
EROWS = E // SUB

_MESH = dict(core_axis_name="c", subcore_axis_name="s", num_cores=NC,
             num_subcores=NS)


def _i32(v):
    return jnp.int32(v)


def _fori(n, body):
    lax.fori_loop(jnp.int32(0), jnp.int32(n), body, jnp.int32(0))


def _zero_fill(ref, nrows, width):
    def body(j, carry):
        for k in range(width // 16):
            ref[j, pl.ds(k * 16, 16)] = jnp.zeros((16,), jnp.float32)
        return carry
    _fori(nrows, body)


def _count_body(dst2, et2, seg2, cnt_a, cnt_b, cntsh, dstv, etv, segv,
                ones, zb, sem):
    core = lax.axis_index("c")
    sub = lax.axis_index("s")
    _zero_fill(zb, 500, 16)
    def ones_body(j, carry):
        ones[j, :] = jnp.ones((16,), jnp.float32)
        return carry
    _fori(SUB, ones_body)
    r0 = sub * _i32(SEGROWS // NS)
    for k in range(SEGROWS // NS // 500):
        pltpu.sync_copy(zb, cntsh.at[pl.ds(r0 + k * 500, 500)])
    plsc.subcore_barrier()

    wid = core * NS + sub
    rows_per_w = EROWS // (NC * NS)
    n_outer = rows_per_w // TILE

    def outer(i, carry):
        rb = wid * _i32(rows_per_w) + i * _i32(TILE)
        pltpu.sync_copy(dst2.at[pl.ds(rb, TILE)], dstv)
        pltpu.sync_copy(et2.at[pl.ds(rb, TILE)], etv)

        def comp(j, c2):
            for k in range(SUB // 16):
                sl = pl.ds(k * 16, 16)
                segv[j, sl] = etv[j, sl] * _i32(N) + dstv[j, sl]
            return c2
        _fori(TILE, comp)
        pltpu.sync_copy(segv, seg2.at[pl.ds(rb, TILE)])
        cps = [pltpu.async_copy(ones, cntsh.at[segv.at[_i32(j)]], sem, add=True)
               for j in range(TILE)]
        for cp in cps:
            cp.wait()
        return carry
    _fori(n_outer, outer)
    plsc.subcore_barrier()

    stripe = SEGROWS // NS
    @pl.when(core == 0)
    def _():
        pltpu.sync_copy(cntsh.at[pl.ds(r0, stripe)],
                        cnt_a.at[pl.ds(r0, stripe)])
    @pl.when(core == 1)
    def _():
        pltpu.sync_copy(cntsh.at[pl.ds(r0, stripe)],
                        cnt_b.at[pl.ds(r0, stripe)])


@jax.jit
def _count_kernel(dst2, et2):
    mesh = plsc.VectorSubcoreMesh(**_MESH)
    return pl.kernel(
        _count_body,
        out_type=[
            jax.ShapeDtypeStruct((EROWS, SUB), jnp.int32),
            jax.ShapeDtypeStruct((SEGROWS, 16), jnp.float32),
            jax.ShapeDtypeStruct((SEGROWS, 16), jnp.float32),
        ],
        mesh=mesh,
        compiler_params=pltpu.CompilerParams(use_tc_tiling_on_sc=False),
        scratch_types=[
            pltpu.VMEM_SHARED((SEGROWS, 16), jnp.float32),
            pltpu.VMEM((TILE, SUB), jnp.int32),
            pltpu.VMEM((TILE, SUB), jnp.int32),
            pltpu.VMEM((TILE, SUB), jnp.int32),
            pltpu.VMEM((SUB, 16), jnp.float32),
            pltpu.VMEM((500, 16), jnp.float32),
            pltpu.SemaphoreType.DMA,
        ],
    )(dst2, et2)


def _agg_body(t0, t1, t2, t3, src2, seg2, a0, a1, a2, a3,
              accsh, sidx, segv, rows, zb, gsem, ssem):
    core = lax.axis_index("c")
    sub = lax.axis_index("s")
    _zero_fill(zb, 250, CW)
    tables = [t0, t1, t2, t3]
    outs = [a0, a1, a2, a3]
    r0 = sub * _i32(SEGROWS // NS)
    stripe = SEGROWS // NS
    rows_per_sub = EROWS // NS
    n_outer = rows_per_sub // TILE_A

    def scatter_pass(table):
        def outer(i, carry):
            rb = sub * _i32(rows_per_sub) + i * _i32(TILE_A)
            pltpu.sync_copy(src2.at[pl.ds(rb, TILE_A)], sidx)
            pltpu.sync_copy(seg2.at[pl.ds(rb, TILE_A)], segv)
            gps = [pltpu.async_copy(table.at[sidx.at[_i32(j)]], rows.at[_i32(j)], gsem)
                   for j in range(TILE_A)]
            sps = []
            for j in range(TILE_A):
                gps[j].wait()
                sps.append(pltpu.async_copy(rows.at[_i32(j)], accsh.at[segv.at[_i32(j)]],
                                            ssem, add=True))
            for cp in sps:
                cp.wait()
            return carry
        _fori(n_outer, outer)

    for p in range(NCH // NC):
        for k in range(stripe // 250):
            pltpu.sync_copy(zb, accsh.at[pl.ds(r0 + k * 250, 250)])
        plsc.subcore_barrier()
        for ch in (p, NCH // NC + p):
            @pl.when(core == ch // (NCH // NC))
            def _(ch=ch):
                scatter_pass(tables[ch])
        plsc.subcore_barrier()
        for ch in (p, NCH // NC + p):
            @pl.when(core == ch // (NCH // NC))
            def _(ch=ch):
                pltpu.sync_copy(accsh.at[pl.ds(r0, stripe)],
                                outs[ch].at[pl.ds(r0, stripe)])


@jax.jit
def _agg_kernel(t0, t1, t2, t3, src2, seg2):
    mesh = plsc.VectorSubcoreMesh(**_MESH)
    out = jax.ShapeDtypeStruct((SEGROWS, CW), jnp.float32)
    return pl.kernel(
        _agg_body,
        out_type=[out, out, out, out],
        mesh=mesh,
        compiler_params=pltpu.CompilerParams(use_tc_tiling_on_sc=False),
        scratch_types=[
            pltpu.VMEM_SHARED((SEGROWS, CW), jnp.float32),
            pltpu.VMEM((TILE_A, SUB), jnp.int32),
            pltpu.VMEM((TILE_A, SUB), jnp.int32),
            pltpu.VMEM((TILE_A, SUB, CW), jnp.float32),
            pltpu.VMEM((250, CW), jnp.float32),
            pltpu.SemaphoreType.DMA,
            pltpu.SemaphoreType.DMA,
        ],
    )(t0, t1, t2, t3, src2, seg2)


BN = 200


def _combine_body(a0, a1, a2, a3, cnt, xr, wp, wroot, b, h, hc):
    cntarr = cnt[...]
    cntb = jnp.maximum(cntarr[0] + cntarr[1], 1.0)
    acc = jnp.dot(xr[...], wroot[...], preferred_element_type=jnp.float32)
    arefs = [a0, a1, a2, a3]
    wparr = wp[...]
    for c in range(NCH):
        ablk = arefs[c][...]
        for r in range(R):
            m = ablk[r] / cntb[:, r:r + 1]
            acc = acc + jnp.dot(m, wparr[c, r],
                                preferred_element_type=jnp.float32)
    out = jnp.maximum(acc + b[...], 0.0)
    h[...] = out
    for c in range(NCH):
        hc[c] = out[:, c * CW:(c + 1) * CW]


@jax.jit
def _combine_kernel(a0, a1, a2, a3, cnt, xin, wp, wroot, b):
    grid = (N // BN,)
    z = lambda: jnp.int32(0)
    ablock = pl.BlockSpec((R, BN, CW), lambda i: (z(), i, z()))
    return pl.pallas_call(
        _combine_body,
        grid=grid,
        in_specs=[
            ablock, ablock, ablock, ablock,
            pl.BlockSpec((2, BN, R), lambda i: (z(), i, z())),
            pl.BlockSpec((BN, D), lambda i: (i, z())),
            pl.BlockSpec((NCH, R, CW, D), lambda i: (z(), z(), z(), z())),
            pl.BlockSpec((D, D), lambda i: (z(), z())),
            pl.BlockSpec((1, D), lambda i: (z(), z())),
        ],
        out_specs=[
            pl.BlockSpec((BN, D), lambda i: (i, z())),
            pl.BlockSpec((NCH, BN, CW), lambda i: (z(), i, z())),
        ],
        out_shape=[
            jax.ShapeDtypeStruct((N, D), jnp.float32),
            jax.ShapeDtypeStruct((NCH, N, CW), jnp.float32),
        ],
    )(a0, a1, a2, a3, cnt, xin, wp, wroot, b)


def _layer(tables, src2, seg2, cnt, xin, Wr, Wroot, b):
    a = _agg_kernel(tables[0], tables[1], tables[2], tables[3], src2, seg2)
    a = [ai.reshape(R, N, CW) for ai in a]
    wp = Wr.reshape(R, NCH, CW, D).transpose(1, 0, 2, 3)
    h, hcat = _combine_kernel(a[0], a[1], a[2], a[3], cnt, xin,
                              wp.astype(jnp.float32),
                              Wroot.astype(jnp.float32),
                              b.reshape(1, D).astype(jnp.float32))
    return h, [hcat[c] for c in range(NCH)]


def kernel(x, edge_index, edge_attr, Wr1, Wroot1, b1, Wr2, Wroot2, b2):
    x = x.astype(jnp.float32)
    src2 = edge_index[0].astype(jnp.int32).reshape(EROWS, SUB)
    dst2 = edge_index[1].astype(jnp.int32).reshape(EROWS, SUB)
    et2 = edge_attr[:, 0].astype(jnp.int32).reshape(EROWS, SUB)

    seg2, cnt_a, cnt_b = _count_kernel(dst2, et2)
    cnt = jnp.stack([cnt_a[:, 0].reshape(R, N).T, cnt_b[:, 0].reshape(R, N).T])

    tables1 = [x[:, c * CW:(c + 1) * CW] for c in range(NCH)]
    h1, tables2 = _layer(tables1, src2, seg2, cnt, x, Wr1, Wroot1, b1)
    h2, _ = _layer(tables2, src2, seg2, cnt, h1, Wr2, Wroot2, b2)
    return jnp.stack([h1, h2]).astype(jnp.float64)


# speedup vs baseline: 177.9225x; 1.0587x over previous
import functools

import jax
import jax.numpy as jnp
from jax import lax
from jax.experimental import pallas as pl
from jax.experimental.pallas import tpu as pltpu
from jax.experimental.pallas import tpu_sc as plsc

N = 10000
E = 320000
D = 128
R = 4
NC = 2
NS = 16
CW = 32
NCH = D // CW
SUB = 80
TILE = 25
TILE_A = 25
GRP = 5
SEGROWS = R * N
EROWS = E // SUB

_MESH = dict(core_axis_name="c", subcore_axis_name="s", num_cores=NC,
             num_subcores=NS)


def _i32(v):
    return jnp.int32(v)


def _fori(n, body):
    lax.fori_loop(jnp.int32(0), jnp.int32(n), body, jnp.int32(0))


def _zero_fill(ref, nrows, width):
    def body(j, carry):
        for k in range(width // 16):
            ref[j, pl.ds(k * 16, 16)] = jnp.zeros((16,), jnp.float32)
        return carry
    _fori(nrows, body)


def _count_body(dst2, et2, seg2, cnt_a, cnt_b, cntsh, dstv, etv, segv,
                ones, zb, sem):
    core = lax.axis_index("c")
    sub = lax.axis_index("s")
    _zero_fill(zb, 500, 16)
    def ones_body(j, carry):
        ones[j, :] = jnp.ones((16,), jnp.float32)
        return carry
    _fori(SUB, ones_body)
    r0 = sub * _i32(SEGROWS // NS)
    for k in range(SEGROWS // NS // 500):
        pltpu.sync_copy(zb, cntsh.at[pl.ds(r0 + k * 500, 500)])
    plsc.subcore_barrier()

    wid = core * NS + sub
    rows_per_w = EROWS // (NC * NS)
    n_outer = rows_per_w // TILE

    def outer(i, carry):
        rb = wid * _i32(rows_per_w) + i * _i32(TILE)
        pltpu.sync_copy(dst2.at[pl.ds(rb, TILE)], dstv)
        pltpu.sync_copy(et2.at[pl.ds(rb, TILE)], etv)

        def comp(j, c2):
            for k in range(SUB // 16):
                sl = pl.ds(k * 16, 16)
                segv[j, sl] = etv[j, sl] * _i32(N) + dstv[j, sl]
            return c2
        _fori(TILE, comp)
        pltpu.sync_copy(segv, seg2.at[pl.ds(rb, TILE)])
        cps = [pltpu.async_copy(ones, cntsh.at[segv.at[_i32(j)]], sem, add=True)
               for j in range(TILE)]
        for cp in cps:
            cp.wait()
        return carry
    _fori(n_outer, outer)
    plsc.subcore_barrier()

    stripe = SEGROWS // NS
    @pl.when(core == 0)
    def _():
        pltpu.sync_copy(cntsh.at[pl.ds(r0, stripe)],
                        cnt_a.at[pl.ds(r0, stripe)])
    @pl.when(core == 1)
    def _():
        pltpu.sync_copy(cntsh.at[pl.ds(r0, stripe)],
                        cnt_b.at[pl.ds(r0, stripe)])


@jax.jit
def _count_kernel(dst2, et2):
    mesh = plsc.VectorSubcoreMesh(**_MESH)
    return pl.kernel(
        _count_body,
        out_type=[
            jax.ShapeDtypeStruct((EROWS, SUB), jnp.int32),
            jax.ShapeDtypeStruct((SEGROWS, 16), jnp.float32),
            jax.ShapeDtypeStruct((SEGROWS, 16), jnp.float32),
        ],
        mesh=mesh,
        compiler_params=pltpu.CompilerParams(use_tc_tiling_on_sc=False),
        scratch_types=[
            pltpu.VMEM_SHARED((SEGROWS, 16), jnp.float32),
            pltpu.VMEM((TILE, SUB), jnp.int32),
            pltpu.VMEM((TILE, SUB), jnp.int32),
            pltpu.VMEM((TILE, SUB), jnp.int32),
            pltpu.VMEM((SUB, 16), jnp.float32),
            pltpu.VMEM((500, 16), jnp.float32),
            pltpu.SemaphoreType.DMA,
        ],
    )(dst2, et2)


def _agg_body(t0, t1, t2, t3, src2, seg2, a0, a1, a2, a3,
              accsh, sidx, segv, rows, zb, isem, gsem, ssem):
    core = lax.axis_index("c")
    sub = lax.axis_index("s")
    _zero_fill(zb, 250, CW)
    tables = [t0, t1, t2, t3]
    outs = [a0, a1, a2, a3]
    r0 = sub * _i32(SEGROWS // NS)
    stripe = SEGROWS // NS
    rows_per_sub = EROWS // NS
    n_outer = rows_per_sub // TILE_A

    def scatter_pass(table):
        idx_cps = {}

        def fire_idx(i):
            bb = _i32(i % 2)
            rb = sub * _i32(rows_per_sub) + _i32(i * TILE_A)
            idx_cps[i] = (
                pltpu.async_copy(src2.at[pl.ds(rb, TILE_A)], sidx.at[bb], isem),
                pltpu.async_copy(seg2.at[pl.ds(rb, TILE_A)], segv.at[bb], isem),
            )

        fire_idx(0)
        pend = {}
        gc = 0
        ngrp = TILE_A // GRP
        for i in range(n_outer):
            b = _i32(i % 2)
            for cp in idx_cps.pop(i):
                cp.wait()
            for g in range(ngrp):
                if g == 2 and i + 1 < n_outer:
                    fire_idx(i + 1)
                half = gc % 2
                if gc >= 2:
                    for cp in pend.pop(gc - 2):
                        cp.wait()
                gps = []
                for k in range(GRP):
                    j = _i32(g * GRP + k)
                    slot = _i32(half * GRP + k)
                    gps.append(pltpu.async_copy(table.at[sidx.at[b, j]],
                                                rows.at[slot], gsem))
                sps = []
                for k in range(GRP):
                    j = _i32(g * GRP + k)
                    slot = _i32(half * GRP + k)
                    gps[k].wait()
                    sps.append(pltpu.async_copy(rows.at[slot],
                                                accsh.at[segv.at[b, j]],
                                                ssem, add=True))
                pend[gc] = sps
                gc += 1
        for gi in sorted(pend):
            for cp in pend[gi]:
                cp.wait()

    for p in range(NCH // NC):
        for k in range(stripe // 250):
            pltpu.sync_copy(zb, accsh.at[pl.ds(r0 + k * 250, 250)])
        plsc.subcore_barrier()
        for ch in (p, NCH // NC + p):
            @pl.when(core == ch // (NCH // NC))
            def _(ch=ch):
                scatter_pass(tables[ch])
        plsc.subcore_barrier()
        for ch in (p, NCH // NC + p):
            @pl.when(core == ch // (NCH // NC))
            def _(ch=ch):
                pltpu.sync_copy(accsh.at[pl.ds(r0, stripe)],
                                outs[ch].at[pl.ds(r0, stripe)])


@jax.jit
def _agg_kernel(t0, t1, t2, t3, src2, seg2):
    mesh = plsc.VectorSubcoreMesh(**_MESH)
    out = jax.ShapeDtypeStruct((SEGROWS, CW), jnp.float32)
    return pl.kernel(
        _agg_body,
        out_type=[out, out, out, out],
        mesh=mesh,
        compiler_params=pltpu.CompilerParams(use_tc_tiling_on_sc=False),
        scratch_types=[
            pltpu.VMEM_SHARED((SEGROWS, CW), jnp.float32),
            pltpu.VMEM((2, TILE_A, SUB), jnp.int32),
            pltpu.VMEM((2, TILE_A, SUB), jnp.int32),
            pltpu.VMEM((2 * GRP, SUB, CW), jnp.float32),
            pltpu.VMEM((250, CW), jnp.float32),
            pltpu.SemaphoreType.DMA,
            pltpu.SemaphoreType.DMA,
            pltpu.SemaphoreType.DMA,
        ],
    )(t0, t1, t2, t3, src2, seg2)


BN = 200


def _combine_body(a0, a1, a2, a3, cnt, xr, wfull, wroot, b, h, hc):
    cntarr = cnt[...]
    rec = 1.0 / jnp.maximum(cntarr[0] + cntarr[1], 1.0)
    acc = jnp.dot(xr[...], wroot[...], preferred_element_type=jnp.float32)
    arefs = [a0, a1, a2, a3]
    pieces = []
    for r in range(R):
        mr = jnp.concatenate([arefs[c][r] for c in range(NCH)], axis=1)
        pieces.append(mr * rec[:, r:r + 1])
    m = jnp.concatenate(pieces, axis=1)
    acc = acc + jnp.dot(m, wfull[...], preferred_element_type=jnp.float32)
    out = jnp.maximum(acc + b[...], 0.0)
    h[...] = out
    for c in range(NCH):
        hc[c] = out[:, c * CW:(c + 1) * CW]


@jax.jit
def _combine_kernel(a0, a1, a2, a3, cnt, xin, wfull, wroot, b):
    grid = (N // BN,)
    z = lambda: jnp.int32(0)
    ablock = pl.BlockSpec((R, BN, CW), lambda i: (z(), i, z()))
    return pl.pallas_call(
        _combine_body,
        grid=grid,
        in_specs=[
            ablock, ablock, ablock, ablock,
            pl.BlockSpec((2, BN, R), lambda i: (z(), i, z())),
            pl.BlockSpec((BN, D), lambda i: (i, z())),
            pl.BlockSpec((R * D, D), lambda i: (z(), z())),
            pl.BlockSpec((D, D), lambda i: (z(), z())),
            pl.BlockSpec((1, D), lambda i: (z(), z())),
        ],
        out_specs=[
            pl.BlockSpec((BN, D), lambda i: (i, z())),
            pl.BlockSpec((NCH, BN, CW), lambda i: (z(), i, z())),
        ],
        out_shape=[
            jax.ShapeDtypeStruct((N, D), jnp.float32),
            jax.ShapeDtypeStruct((NCH, N, CW), jnp.float32),
        ],
    )(a0, a1, a2, a3, cnt, xin, wfull, wroot, b)


def _layer(tables, src2, seg2, cnt, xin, Wr, Wroot, b):
    a = _agg_kernel(tables[0], tables[1], tables[2], tables[3], src2, seg2)
    a = [ai.reshape(R, N, CW) for ai in a]
    wfull = Wr.astype(jnp.float32).reshape(R * D, D)
    h, hcat = _combine_kernel(a[0], a[1], a[2], a[3], cnt, xin,
                              wfull,
                              Wroot.astype(jnp.float32),
                              b.reshape(1, D).astype(jnp.float32))
    return h, [hcat[c] for c in range(NCH)]


def kernel(x, edge_index, edge_attr, Wr1, Wroot1, b1, Wr2, Wroot2, b2):
    x = x.astype(jnp.float32)
    src2 = edge_index[0].astype(jnp.int32).reshape(EROWS, SUB)
    dst2 = edge_index[1].astype(jnp.int32).reshape(EROWS, SUB)
    et2 = edge_attr[:, 0].astype(jnp.int32).reshape(EROWS, SUB)

    seg2, cnt_a, cnt_b = _count_kernel(dst2, et2)
    cnt = jnp.stack([cnt_a[:, 0].reshape(R, N).T, cnt_b[:, 0].reshape(R, N).T])

    tables1 = [x[:, c * CW:(c + 1) * CW] for c in range(NCH)]
    h1, tables2 = _layer(tables1, src2, seg2, cnt, x, Wr1, Wroot1, b1)
    h2, _ = _layer(tables2, src2, seg2, cnt, h1, Wr2, Wroot2, b2)
    return jnp.stack([h1, h2]).astype(jnp.float64)


# speedup vs baseline: 184.9067x; 1.0393x over previous
import functools

import jax
import jax.numpy as jnp
from jax import lax
from jax.experimental import pallas as pl
from jax.experimental.pallas import tpu as pltpu
from jax.experimental.pallas import tpu_sc as plsc

N = 10000
E = 320000
D = 128
R = 4
NC = 2
NS = 16
CW = 32
NCH = D // CW
SUB = 80
TILE = 25
TILE_A = 25
GRP = 5
SEGROWS = R * N
EROWS = E // SUB

_MESH = dict(core_axis_name="c", subcore_axis_name="s", num_cores=NC,
             num_subcores=NS)


def _i32(v):
    return jnp.int32(v)


def _fori(n, body):
    lax.fori_loop(jnp.int32(0), jnp.int32(n), body, jnp.int32(0))


def _zero_fill(ref, nrows, width):
    def body(j, carry):
        for k in range(width // 16):
            ref[j, pl.ds(k * 16, 16)] = jnp.zeros((16,), jnp.float32)
        return carry
    _fori(nrows, body)


def _count_body(dst2, et2, seg2, cnt_a, cnt_b, cntsh, dstv, etv, segv,
                ones, zb, sem):
    core = lax.axis_index("c")
    sub = lax.axis_index("s")
    _zero_fill(zb, 500, 16)
    def ones_body(j, carry):
        ones[j, :] = jnp.ones((16,), jnp.float32)
        return carry
    _fori(SUB, ones_body)
    r0 = sub * _i32(SEGROWS // NS)
    for k in range(SEGROWS // NS // 500):
        pltpu.sync_copy(zb, cntsh.at[pl.ds(r0 + k * 500, 500)])
    plsc.subcore_barrier()

    wid = core * NS + sub
    rows_per_w = EROWS // (NC * NS)
    n_outer = rows_per_w // TILE

    def outer(i, carry):
        rb = wid * _i32(rows_per_w) + i * _i32(TILE)
        pltpu.sync_copy(dst2.at[pl.ds(rb, TILE)], dstv)
        pltpu.sync_copy(et2.at[pl.ds(rb, TILE)], etv)

        def comp(j, c2):
            for k in range(SUB // 16):
                sl = pl.ds(k * 16, 16)
                segv[j, sl] = etv[j, sl] * _i32(N) + dstv[j, sl]
            return c2
        _fori(TILE, comp)
        pltpu.sync_copy(segv, seg2.at[pl.ds(rb, TILE)])
        cps = [pltpu.async_copy(ones, cntsh.at[segv.at[_i32(j)]], sem, add=True)
               for j in range(TILE)]
        for cp in cps:
            cp.wait()
        return carry
    _fori(n_outer, outer)
    plsc.subcore_barrier()

    stripe = SEGROWS // NS
    @pl.when(core == 0)
    def _():
        pltpu.sync_copy(cntsh.at[pl.ds(r0, stripe)],
                        cnt_a.at[pl.ds(r0, stripe)])
    @pl.when(core == 1)
    def _():
        pltpu.sync_copy(cntsh.at[pl.ds(r0, stripe)],
                        cnt_b.at[pl.ds(r0, stripe)])


@jax.jit
def _count_kernel(dst2, et2):
    mesh = plsc.VectorSubcoreMesh(**_MESH)
    return pl.kernel(
        _count_body,
        out_type=[
            jax.ShapeDtypeStruct((EROWS, SUB), jnp.int32),
            jax.ShapeDtypeStruct((SEGROWS, 16), jnp.float32),
            jax.ShapeDtypeStruct((SEGROWS, 16), jnp.float32),
        ],
        mesh=mesh,
        compiler_params=pltpu.CompilerParams(use_tc_tiling_on_sc=False),
        scratch_types=[
            pltpu.VMEM_SHARED((SEGROWS, 16), jnp.float32),
            pltpu.VMEM((TILE, SUB), jnp.int32),
            pltpu.VMEM((TILE, SUB), jnp.int32),
            pltpu.VMEM((TILE, SUB), jnp.int32),
            pltpu.VMEM((SUB, 16), jnp.float32),
            pltpu.VMEM((500, 16), jnp.float32),
            pltpu.SemaphoreType.DMA,
        ],
    )(dst2, et2)


def _agg_body(t0, t1, src2, seg2, a0, a1,
              accsh, sidx, segv, rows, zb, isem, gsem, ssem):
    core = lax.axis_index("c")
    sub = lax.axis_index("s")
    _zero_fill(zb, 250, CW)
    tables = [t0, t1]
    outs = [a0, a1]
    r0 = sub * _i32(SEGROWS // NS)
    stripe = SEGROWS // NS
    rows_per_sub = EROWS // NS
    n_outer = rows_per_sub // TILE_A

    def scatter_pass(table):
        idx_cps = {}

        def fire_idx(i):
            bb = _i32(i % 2)
            rb = sub * _i32(rows_per_sub) + _i32(i * TILE_A)
            idx_cps[i] = (
                pltpu.async_copy(src2.at[pl.ds(rb, TILE_A)], sidx.at[bb], isem),
                pltpu.async_copy(seg2.at[pl.ds(rb, TILE_A)], segv.at[bb], isem),
            )

        fire_idx(0)
        pend = {}
        gc = 0
        ngrp = TILE_A // GRP
        for i in range(n_outer):
            b = _i32(i % 2)
            for cp in idx_cps.pop(i):
                cp.wait()
            for g in range(ngrp):
                if g == 2 and i + 1 < n_outer:
                    fire_idx(i + 1)
                half = gc % 2
                if gc >= 2:
                    for cp in pend.pop(gc - 2):
                        cp.wait()
                gps = []
                for k in range(GRP):
                    j = _i32(g * GRP + k)
                    slot = _i32(half * GRP + k)
                    gps.append(pltpu.async_copy(table.at[sidx.at[b, j]],
                                                rows.at[slot], gsem))
                sps = []
                for k in range(GRP):
                    j = _i32(g * GRP + k)
                    slot = _i32(half * GRP + k)
                    gps[k].wait()
                    sps.append(pltpu.async_copy(rows.at[slot],
                                                accsh.at[segv.at[b, j]],
                                                ssem, add=True))
                pend[gc] = sps
                gc += 1
        for gi in sorted(pend):
            for cp in pend[gi]:
                cp.wait()

    for k in range(stripe // 250):
        pltpu.sync_copy(zb, accsh.at[pl.ds(r0 + k * 250, 250)])
    plsc.subcore_barrier()
    for ch in range(2):
        @pl.when(core == ch)
        def _(ch=ch):
            scatter_pass(tables[ch])
    plsc.subcore_barrier()
    for ch in range(2):
        @pl.when(core == ch)
        def _(ch=ch):
            pltpu.sync_copy(accsh.at[pl.ds(r0, stripe)],
                            outs[ch].at[pl.ds(r0, stripe)])


@jax.jit
def _agg_kernel(t0, t1, src2, seg2):
    mesh = plsc.VectorSubcoreMesh(**_MESH)
    out = jax.ShapeDtypeStruct((SEGROWS, CW), jnp.float32)
    return pl.kernel(
        _agg_body,
        out_type=[out, out],
        mesh=mesh,
        compiler_params=pltpu.CompilerParams(use_tc_tiling_on_sc=False),
        scratch_types=[
            pltpu.VMEM_SHARED((SEGROWS, CW), jnp.float32),
            pltpu.VMEM((2, TILE_A, SUB), jnp.int32),
            pltpu.VMEM((2, TILE_A, SUB), jnp.int32),
            pltpu.VMEM((2 * GRP, SUB, CW), jnp.float32),
            pltpu.VMEM((250, CW), jnp.float32),
            pltpu.SemaphoreType.DMA,
            pltpu.SemaphoreType.DMA,
            pltpu.SemaphoreType.DMA,
        ],
    )(t0, t1, src2, seg2)


BN = 200


def _combine_body(a0, a1, a2, a3, cnt, xr, wfull, wroot, b, h, hc):
    cntarr = cnt[...]
    rec = 1.0 / jnp.maximum(cntarr[0] + cntarr[1], 1.0)
    acc = jnp.dot(xr[...], wroot[...], preferred_element_type=jnp.float32)
    arefs = [a0, a1, a2, a3]
    pieces = []
    for r in range(R):
        mr = jnp.concatenate([arefs[c][r] for c in range(NCH)], axis=1)
        pieces.append(mr * rec[:, r:r + 1])
    m = jnp.concatenate(pieces, axis=1)
    acc = acc + jnp.dot(m, wfull[...], preferred_element_type=jnp.float32)
    out = jnp.maximum(acc + b[...], 0.0)
    h[...] = out
    for c in range(NCH):
        hc[c] = out[:, c * CW:(c + 1) * CW]


@jax.jit
def _combine_kernel(a0, a1, a2, a3, cnt, xin, wfull, wroot, b):
    grid = (N // BN,)
    z = lambda: jnp.int32(0)
    ablock = pl.BlockSpec((R, BN, CW), lambda i: (z(), i, z()))
    return pl.pallas_call(
        _combine_body,
        grid=grid,
        in_specs=[
            ablock, ablock, ablock, ablock,
            pl.BlockSpec((2, BN, R), lambda i: (z(), i, z())),
            pl.BlockSpec((BN, D), lambda i: (i, z())),
            pl.BlockSpec((R * D, D), lambda i: (z(), z())),
            pl.BlockSpec((D, D), lambda i: (z(), z())),
            pl.BlockSpec((1, D), lambda i: (z(), z())),
        ],
        out_specs=[
            pl.BlockSpec((BN, D), lambda i: (i, z())),
            pl.BlockSpec((NCH, BN, CW), lambda i: (z(), i, z())),
        ],
        out_shape=[
            jax.ShapeDtypeStruct((N, D), jnp.float32),
            jax.ShapeDtypeStruct((NCH, N, CW), jnp.float32),
        ],
    )(a0, a1, a2, a3, cnt, xin, wfull, wroot, b)


def _layer(tables, src2, seg2, cnt, xin, Wr, Wroot, b):
    a01 = _agg_kernel(tables[0], tables[1], src2, seg2)
    a23 = _agg_kernel(tables[2], tables[3], src2, seg2)
    a = [ai.reshape(R, N, CW) for ai in (*a01, *a23)]
    wfull = Wr.astype(jnp.float32).reshape(R * D, D)
    h, hcat = _combine_kernel(a[0], a[1], a[2], a[3], cnt, xin,
                              wfull,
                              Wroot.astype(jnp.float32),
                              b.reshape(1, D).astype(jnp.float32))
    return h, [hcat[c] for c in range(NCH)]


def kernel(x, edge_index, edge_attr, Wr1, Wroot1, b1, Wr2, Wroot2, b2):
    x = x.astype(jnp.float32)
    src2 = edge_index[0].astype(jnp.int32).reshape(EROWS, SUB)
    dst2 = edge_index[1].astype(jnp.int32).reshape(EROWS, SUB)
    et2 = edge_attr[:, 0].astype(jnp.int32).reshape(EROWS, SUB)

    seg2, cnt_a, cnt_b = _count_kernel(dst2, et2)
    cnt = jnp.stack([cnt_a[:, 0].reshape(R, N).T, cnt_b[:, 0].reshape(R, N).T])

    tables1 = [x[:, c * CW:(c + 1) * CW] for c in range(NCH)]
    h1, tables2 = _layer(tables1, src2, seg2, cnt, x, Wr1, Wroot1, b1)
    h2, _ = _layer(tables2, src2, seg2, cnt, h1, Wr2, Wroot2, b2)
    return jnp.stack([h1, h2]).astype(jnp.float64)


# speedup vs baseline: 185.7829x; 1.0047x over previous
import jax
import jax.numpy as jnp
from jax import lax
from jax.experimental import pallas as pl
from jax.experimental.pallas import tpu as pltpu
from jax.experimental.pallas import tpu_sc as plsc

N = 10000
E = 320000
D = 128
R = 4
NC = 2
NS = 16
CW = 32
NCH = D // CW
STREAM = 400
SEGROWS = R * N
EPC = E // (NC * NS)
EPA = E // NS

_MESH = dict(core_axis_name="c", subcore_axis_name="s", num_cores=NC,
             num_subcores=NS)


def _i32(v):
    return jnp.int32(v)


def _fori(n, body):
    lax.fori_loop(jnp.int32(0), jnp.int32(n), body, jnp.int32(0))


def _zero_fill(ref, nrows, width):
    def body(j, carry):
        for k in range(width // 16):
            ref[j, pl.ds(k * 16, 16)] = jnp.zeros((16,), jnp.float32)
        return carry
    _fori(nrows, body)


CB = 2000


def _count_body(dst1, et1, seg1, cnt_a, cnt_b, cntsh, dstv, etv, segv,
                ones, zb, sem):
    core = lax.axis_index("c")
    sub = lax.axis_index("s")
    _zero_fill(zb, 500, 16)
    def ones_body(j, carry):
        ones[j, :] = jnp.ones((16,), jnp.float32)
        return carry
    _fori(CB, ones_body)
    r0 = sub * _i32(SEGROWS // NS)
    for k in range(SEGROWS // NS // 500):
        pltpu.sync_copy(zb, cntsh.at[pl.ds(r0 + k * 500, 500)])
    plsc.subcore_barrier()

    wid = core * NS + sub
    n_outer = EPC // CB

    def outer(i, carry):
        eb = wid * _i32(EPC) + i * _i32(CB)
        pltpu.sync_copy(dst1.at[pl.ds(eb, CB)], dstv)
        pltpu.sync_copy(et1.at[pl.ds(eb, CB)], etv)

        def comp(j, c2):
            sl = pl.ds(j * 16, 16)
            segv[sl] = etv[sl] * _i32(N) + dstv[sl]
            return c2
        _fori(CB // 16, comp)
        pltpu.sync_copy(segv, seg1.at[pl.ds(eb, CB)])
        pltpu.async_copy(ones, cntsh.at[segv], sem, add=True).wait()
        return carry
    _fori(n_outer, outer)
    plsc.subcore_barrier()

    stripe = SEGROWS // NS
    @pl.when(core == 0)
    def _():
        pltpu.sync_copy(cntsh.at[pl.ds(r0, stripe)],
                        cnt_a.at[pl.ds(r0, stripe)])
    @pl.when(core == 1)
    def _():
        pltpu.sync_copy(cntsh.at[pl.ds(r0, stripe)],
                        cnt_b.at[pl.ds(r0, stripe)])


@jax.jit
def _count_kernel(dst1, et1):
    mesh = plsc.VectorSubcoreMesh(**_MESH)
    return pl.kernel(
        _count_body,
        out_type=[
            jax.ShapeDtypeStruct((E,), jnp.int32),
            jax.ShapeDtypeStruct((SEGROWS, 16), jnp.float32),
            jax.ShapeDtypeStruct((SEGROWS, 16), jnp.float32),
        ],
        mesh=mesh,
        compiler_params=pltpu.CompilerParams(use_tc_tiling_on_sc=False),
        scratch_types=[
            pltpu.VMEM_SHARED((SEGROWS, 16), jnp.float32),
            pltpu.VMEM((CB,), jnp.int32),
            pltpu.VMEM((CB,), jnp.int32),
            pltpu.VMEM((CB,), jnp.int32),
            pltpu.VMEM((CB, 16), jnp.float32),
            pltpu.VMEM((500, 16), jnp.float32),
            pltpu.SemaphoreType.DMA,
        ],
    )(dst1, et1)


def _agg_body(t0, t1, src1, seg1, a0, a1,
              accsh, sidx, segv, rows, zb, isem, gsem, ssem):
    core = lax.axis_index("c")
    sub = lax.axis_index("s")
    _zero_fill(zb, 250, CW)
    tables = [t0, t1]
    outs = [a0, a1]
    r0 = sub * _i32(SEGROWS // NS)
    stripe = SEGROWS // NS
    n_outer = EPA // STREAM

    def scatter_pass(table):
        idx_cps = {}

        def fire_idx(i):
            bb = _i32(i % 3)
            eb = sub * _i32(EPA) + _i32(i * STREAM)
            idx_cps[i] = (
                pltpu.async_copy(src1.at[pl.ds(eb, STREAM)], sidx.at[bb], isem),
                pltpu.async_copy(seg1.at[pl.ds(eb, STREAM)], segv.at[bb], isem),
            )

        fire_idx(0)
        pend = {}
        for i in range(n_outer):
            b3 = _i32(i % 3)
            b2 = _i32(i % 2)
            for cp in idx_cps.pop(i):
                cp.wait()
            if i >= 2:
                pend.pop(i - 2).wait()
            if i + 1 < n_outer:
                fire_idx(i + 1)
            gcp = pltpu.async_copy(table.at[sidx.at[b3]], rows.at[b2], gsem)
            gcp.wait()
            pend[i] = pltpu.async_copy(rows.at[b2], accsh.at[segv.at[b3]],
                                       ssem, add=True)
        for gi in sorted(pend):
            pend[gi].wait()

    for k in range(stripe // 250):
        pltpu.sync_copy(zb, accsh.at[pl.ds(r0 + k * 250, 250)])
    plsc.subcore_barrier()
    for ch in range(2):
        @pl.when(core == ch)
        def _(ch=ch):
            scatter_pass(tables[ch])
    plsc.subcore_barrier()
    for ch in range(2):
        @pl.when(core == ch)
        def _(ch=ch):
            pltpu.sync_copy(accsh.at[pl.ds(r0, stripe)],
                            outs[ch].at[pl.ds(r0, stripe)])


@jax.jit
def _agg_kernel(t0, t1, src1, seg1):
    mesh = plsc.VectorSubcoreMesh(**_MESH)
    out = jax.ShapeDtypeStruct((SEGROWS, CW), jnp.float32)
    return pl.kernel(
        _agg_body,
        out_type=[out, out],
        mesh=mesh,
        compiler_params=pltpu.CompilerParams(use_tc_tiling_on_sc=False),
        scratch_types=[
            pltpu.VMEM_SHARED((SEGROWS, CW), jnp.float32),
            pltpu.VMEM((3, STREAM), jnp.int32),
            pltpu.VMEM((3, STREAM), jnp.int32),
            pltpu.VMEM((2, STREAM, CW), jnp.float32),
            pltpu.VMEM((250, CW), jnp.float32),
            pltpu.SemaphoreType.DMA,
            pltpu.SemaphoreType.DMA,
            pltpu.SemaphoreType.DMA,
        ],
    )(t0, t1, src1, seg1)


BN = 200


def _combine_body(a0, a1, a2, a3, cnt, xr, wfull, wroot, b, h, hc):
    cntarr = cnt[...]
    rec = 1.0 / jnp.maximum(cntarr[0] + cntarr[1], 1.0)
    acc = jnp.dot(xr[...], wroot[...], preferred_element_type=jnp.float32)
    arefs = [a0, a1, a2, a3]
    pieces = []
    for r in range(R):
        mr = jnp.concatenate([arefs[c][r] for c in range(NCH)], axis=1)
        pieces.append(mr * rec[:, r:r + 1])
    m = jnp.concatenate(pieces, axis=1)
    acc = acc + jnp.dot(m, wfull[...], preferred_element_type=jnp.float32)
    out = jnp.maximum(acc + b[...], 0.0)
    h[...] = out
    for c in range(NCH):
        hc[c] = out[:, c * CW:(c + 1) * CW]


@jax.jit
def _combine_kernel(a0, a1, a2, a3, cnt, xin, wfull, wroot, b):
    grid = (N // BN,)
    z = lambda: jnp.int32(0)
    ablock = pl.BlockSpec((R, BN, CW), lambda i: (z(), i, z()))
    return pl.pallas_call(
        _combine_body,
        grid=grid,
        in_specs=[
            ablock, ablock, ablock, ablock,
            pl.BlockSpec((2, BN, R), lambda i: (z(), i, z())),
            pl.BlockSpec((BN, D), lambda i: (i, z())),
            pl.BlockSpec((R * D, D), lambda i: (z(), z())),
            pl.BlockSpec((D, D), lambda i: (z(), z())),
            pl.BlockSpec((1, D), lambda i: (z(), z())),
        ],
        out_specs=[
            pl.BlockSpec((BN, D), lambda i: (i, z())),
            pl.BlockSpec((NCH, BN, CW), lambda i: (z(), i, z())),
        ],
        out_shape=[
            jax.ShapeDtypeStruct((N, D), jnp.float32),
            jax.ShapeDtypeStruct((NCH, N, CW), jnp.float32),
        ],
    )(a0, a1, a2, a3, cnt, xin, wfull, wroot, b)


def _layer(tables, src1, seg1, cnt, xin, Wr, Wroot, b):
    a01 = _agg_kernel(tables[0], tables[1], src1, seg1)
    a23 = _agg_kernel(tables[2], tables[3], src1, seg1)
    a = [ai.reshape(R, N, CW) for ai in (*a01, *a23)]
    wfull = Wr.astype(jnp.float32).reshape(R * D, D)
    h, hcat = _combine_kernel(a[0], a[1], a[2], a[3], cnt, xin,
                              wfull,
                              Wroot.astype(jnp.float32),
                              b.reshape(1, D).astype(jnp.float32))
    return h, [hcat[c] for c in range(NCH)]


def kernel(x, edge_index, edge_attr, Wr1, Wroot1, b1, Wr2, Wroot2, b2):
    x = x.astype(jnp.float32)
    src1 = edge_index[0].astype(jnp.int32)
    dst1 = edge_index[1].astype(jnp.int32)
    et1 = edge_attr[:, 0].astype(jnp.int32)

    seg1, cnt_a, cnt_b = _count_kernel(dst1, et1)
    cnt = jnp.stack([cnt_a[:, 0].reshape(R, N).T, cnt_b[:, 0].reshape(R, N).T])

    tables1 = [x[:, c * CW:(c + 1) * CW] for c in range(NCH)]
    h1, tables2 = _layer(tables1, src1, seg1, cnt, x, Wr1, Wroot1, b1)
    h2, _ = _layer(tables2, src1, seg1, cnt, h1, Wr2, Wroot2, b2)
    return jnp.stack([h1, h2]).astype(jnp.float64)
